# baseline (speedup 1.0000x reference)
import functools

import jax
import jax.numpy as jnp
from jax import lax
from jax.experimental import pallas as pl
from jax.experimental.pallas import tpu as pltpu
from jax.experimental.pallas import tpu_sc as plsc

R1 = 32
R2 = 32
N1 = 200
N2 = 200
B = 16384

NW = 32
PER_W = B // NW
NBH = PER_W // 128
RG = 4
NRG = R1 // RG
NROUND = NBH * NRG
ROWS = RG * 128
GL = 128


def _sc_kernel(ind_flat, table, r1n):
    mesh = plsc.VectorSubcoreMesh(core_axis_name="c", subcore_axis_name="s")
    nrg = r1n // RG
    nround = NBH * nrg

    @functools.partial(
        pl.kernel,
        mesh=mesh,
        out_type=jax.ShapeDtypeStruct((r1n, R2 // 8, B // 128, 8, 128),
                                      jnp.float32),
        compiler_params=pltpu.CompilerParams(
            use_tc_tiling_on_sc=False, needs_layout_passes=False),
        scratch_types=[
            pltpu.VMEM((PER_W,), jnp.int32),
            pltpu.VMEM((PER_W,), jnp.int32),
            pltpu.VMEM((PER_W,), jnp.int32),
            pltpu.VMEM((2, RG, GL), jnp.int32),
            pltpu.VMEM((2, ROWS, R2), jnp.float32),
            pltpu.VMEM((2, RG, 4, 8, 128), jnp.float32),
            pltpu.SemaphoreType.DMA,
            pltpu.SemaphoreType.DMA,
            pltpu.SemaphoreType.DMA,
            pltpu.SemaphoreType.DMA,
        ],
    )
    def k(ind_hbm, tab_hbm, out_hbm, i0_v, i1_v, j_v, idx_v, rows_v,
          tiles_v, gsem0, gsem1, osem0, osem1):
        wid = lax.axis_index("s") * 2 + lax.axis_index("c")
        base = wid * PER_W
        lane = lax.iota(jnp.int32, 16)
        gsems = (gsem0, gsem1)
        osems = (osem0, osem1)

        pltpu.sync_copy(ind_hbm.at[pl.ds(base, PER_W)], i0_v)
        pltpu.sync_copy(ind_hbm.at[pl.ds(B + base, PER_W)], i1_v)
        for v in range(PER_W // 16):
            sl = pl.ds(v * 16, 16)
            j_v[sl] = i0_v[sl] * N2 + i1_v[sl]

        def build_and_fire(r, s):
            bh_i = r // nrg
            r1g = r % nrg
            for gi in range(RG):
                rbase = (r1g * RG + gi) * (N1 * N2)
                for v in range(GL // 16):
                    idx_v[s, gi, pl.ds(v * 16, 16)] = (
                        j_v[pl.ds(bh_i * 128 + v * 16, 16)] + rbase
                    )
            for gi in range(RG):
                pltpu.async_copy(
                    tab_hbm.at[idx_v.at[s, gi]],
                    rows_v.at[s, pl.ds(gi * GL, GL), :],
                    gsems[s],
                )

        def drain_gathers(s):
            for gi in range(RG):
                pltpu.make_async_copy(
                    tab_hbm.at[idx_v.at[s, gi]],
                    rows_v.at[s, pl.ds(gi * GL, GL), :],
                    gsems[s],
                ).wait()

        def transpose(s):
            rows2 = rows_v.at[s]

            @plsc.parallel_loop(0, RG * 4, unroll=8)
            def m_body(m):
                r1i = m // 4
                r2h = m % 4
                rowbase = r1i * 128 + lane
                for r2l in range(8):
                    colv = jnp.broadcast_to(r2h * 8 + r2l, (16,))
                    for blv in range(8):
                        val = plsc.load_gather(
                            rows2, [rowbase + blv * 16, colv])
                        tiles_v[s, r1i, r2h, r2l, pl.ds(blv * 16, 16)] = val

        def out_copies(r, s):
            bh = wid * NBH + r // nrg
            r1g = r % nrg
            return [
                pltpu.make_async_copy(
                    tiles_v.at[s, r1i],
                    out_hbm.at[r1g * RG + r1i, :, bh],
                    osems[s],
                )
                for r1i in range(RG)
            ]

        def fire_outs(r, s):
            bh = wid * NBH + r // nrg
            r1g = r % nrg
            for r1i in range(RG):
                pltpu.async_copy(
                    tiles_v.at[s, r1i],
                    out_hbm.at[r1g * RG + r1i, :, bh],
                    osems[s],
                )

        def drain_outs(r, s):
            for c in out_copies(r, s):
                c.wait()

        build_and_fire(0, 0)

        def pair_body(q, carry):
            r_odd = 2 * q + 1

            @pl.when(q >= 1)
            def _():
                drain_outs(2 * q - 2, 0)

            build_and_fire(r_odd, 1)
            drain_gathers(0)
            transpose(0)
            fire_outs(2 * q, 0)

            @pl.when(q >= 1)
            def _():
                drain_outs(2 * q - 1, 1)

            @pl.when(q < nround // 2 - 1)
            def _():
                build_and_fire(2 * q + 2, 0)

            drain_gathers(1)
            transpose(1)
            fire_outs(2 * q + 1, 1)
            return carry

        lax.fori_loop(0, nround // 2, pair_body, 0)
        drain_outs(nround - 2, 0)
        drain_outs(nround - 1, 1)

    return k(ind_flat, table)


def kernel(indices, tt_core):
    ind_flat = indices.T.reshape(-1)
    halves = [
        _sc_kernel(ind_flat,
                   tt_core[h * 16:(h + 1) * 16].reshape(16 * N1 * N2, R2), 16)
        for h in (0, 1)
    ]
    out6 = jnp.concatenate(halves, axis=0)
    return out6.transpose(2, 4, 0, 1, 3).reshape(B, R1, R2)


# speedup vs baseline: 1.3698x; 1.3698x over previous
import functools

import jax
import jax.numpy as jnp
from jax import lax
from jax.experimental import pallas as pl
from jax.experimental.pallas import tpu as pltpu
from jax.experimental.pallas import tpu_sc as plsc

R1 = 32
R2 = 32
N1 = 200
N2 = 200
B = 16384

NW = 32
PER_W = B // NW
NBH = PER_W // 128
RG = 4
NRG = R1 // RG
NROUND = NBH * NRG
ROWS = RG * 128
GL = 128


def _sc_kernel(ind_flat, table):
    mesh = plsc.VectorSubcoreMesh(core_axis_name="c", subcore_axis_name="s")

    @functools.partial(
        pl.kernel,
        mesh=mesh,
        out_type=jax.ShapeDtypeStruct((R1, R2 // 8, B // 128, 8, 128),
                                      jnp.float32),
        compiler_params=pltpu.CompilerParams(
            use_tc_tiling_on_sc=False, needs_layout_passes=False),
        scratch_types=[
            pltpu.VMEM((PER_W,), jnp.int32),
            pltpu.VMEM((PER_W,), jnp.int32),
            pltpu.VMEM((PER_W,), jnp.int32),
            pltpu.VMEM((2, RG, GL), jnp.int32),
            pltpu.VMEM((2, ROWS, R2), jnp.float32),
            pltpu.VMEM((2, RG, 4, 8, 128), jnp.float32),
            pltpu.SemaphoreType.DMA,
            pltpu.SemaphoreType.DMA,
            pltpu.SemaphoreType.DMA,
            pltpu.SemaphoreType.DMA,
        ],
    )
    def k(ind_hbm, tab_hbm, out_hbm, i0_v, i1_v, j_v, idx_v, rows_v,
          tiles_v, gsem0, gsem1, osem0, osem1):
        wid = lax.axis_index("s") * 2 + lax.axis_index("c")
        base = wid * PER_W
        lane = lax.iota(jnp.int32, 16)
        gsems = (gsem0, gsem1)
        osems = (osem0, osem1)

        pltpu.sync_copy(ind_hbm.at[pl.ds(base, PER_W)], i0_v)
        pltpu.sync_copy(ind_hbm.at[pl.ds(B + base, PER_W)], i1_v)
        for v in range(PER_W // 16):
            sl = pl.ds(v * 16, 16)
            j_v[sl] = i0_v[sl] * N2 + i1_v[sl]

        def build_and_fire(r, s):
            bh_i = r // NRG
            r1g = r % NRG
            for gi in range(RG):
                rbase = (r1g * RG + gi) * (N1 * N2)
                for v in range(GL // 16):
                    idx_v[s, gi, pl.ds(v * 16, 16)] = (
                        j_v[pl.ds(bh_i * 128 + v * 16, 16)] + rbase
                    )
            for gi in range(RG):
                pltpu.async_copy(
                    tab_hbm.at[idx_v.at[s, gi]],
                    rows_v.at[s, pl.ds(gi * GL, GL), :],
                    gsems[s],
                )

        def drain_gathers(s):
            for gi in range(RG):
                pltpu.make_async_copy(
                    tab_hbm.at[idx_v.at[s, gi]],
                    rows_v.at[s, pl.ds(gi * GL, GL), :],
                    gsems[s],
                ).wait()

        def transpose(s):
            rows2 = rows_v.at[s]

            @plsc.parallel_loop(0, RG * 32, unroll=8)
            def m_body(m):
                r1i = m // 32
                r2 = m % 32
                r2h = r2 // 8
                r2l = r2 % 8
                rowbase = r1i * 128 + lane
                colv = jnp.broadcast_to(r2, (16,))
                for blv in range(8):
                    val = plsc.load_gather(
                        rows2, [rowbase + blv * 16, colv])
                    tiles_v[s, r1i, r2h, r2l, pl.ds(blv * 16, 16)] = val

        def out_copies(r, s):
            bh = wid * NBH + r // NRG
            r1g = r % NRG
            return [
                pltpu.make_async_copy(
                    tiles_v.at[s, r1i],
                    out_hbm.at[r1g * RG + r1i, :, bh],
                    osems[s],
                )
                for r1i in range(RG)
            ]

        def fire_outs(r, s):
            bh = wid * NBH + r // NRG
            r1g = r % NRG
            for r1i in range(RG):
                pltpu.async_copy(
                    tiles_v.at[s, r1i],
                    out_hbm.at[r1g * RG + r1i, :, bh],
                    osems[s],
                )

        def drain_outs(r, s):
            for c in out_copies(r, s):
                c.wait()

        build_and_fire(0, 0)

        def pair_body(q, carry):
            r_odd = 2 * q + 1

            @pl.when(q >= 1)
            def _():
                drain_outs(2 * q - 2, 0)

            build_and_fire(r_odd, 1)
            drain_gathers(0)
            transpose(0)
            fire_outs(2 * q, 0)

            @pl.when(q >= 1)
            def _():
                drain_outs(2 * q - 1, 1)

            @pl.when(q < NROUND // 2 - 1)
            def _():
                build_and_fire(2 * q + 2, 0)

            drain_gathers(1)
            transpose(1)
            fire_outs(2 * q + 1, 1)
            return carry

        lax.fori_loop(0, NROUND // 2, pair_body, 0)
        drain_outs(NROUND - 2, 0)
        drain_outs(NROUND - 1, 1)

    return k(ind_flat, table)


def kernel(indices, tt_core):
    ind_flat = indices.T.reshape(-1)
    table = tt_core.reshape(R1 * N1 * N2, R2)
    out6 = _sc_kernel(ind_flat, table)
    return out6.transpose(2, 4, 0, 1, 3).reshape(B, R1, R2)


# speedup vs baseline: 1.7284x; 1.2617x over previous
import functools

import jax
import jax.numpy as jnp
from jax import lax
from jax.experimental import pallas as pl
from jax.experimental.pallas import tpu as pltpu
from jax.experimental.pallas import tpu_sc as plsc

R1 = 32
R2 = 32
N1 = 200
N2 = 200
B = 16384

NW = 32
PER_W = B // NW
NBH = PER_W // 128
RG = 4
NRG = R1 // RG
NROUND = NBH * NRG
ROWS = RG * 128
GL = 128


def _sc_kernel(ind_flat, table):
    mesh = plsc.VectorSubcoreMesh(core_axis_name="c", subcore_axis_name="s")

    @functools.partial(
        pl.kernel,
        mesh=mesh,
        out_type=jax.ShapeDtypeStruct((R1, R2 // 8, B // 128, 8, 128),
                                      jnp.float32),
        compiler_params=pltpu.CompilerParams(
            use_tc_tiling_on_sc=False, needs_layout_passes=False),
        scratch_types=[
            pltpu.VMEM((PER_W,), jnp.int32),
            pltpu.VMEM((PER_W,), jnp.int32),
            pltpu.VMEM((PER_W,), jnp.int32),
            pltpu.VMEM((2, RG, GL), jnp.int32),
            pltpu.VMEM((2, ROWS, R2), jnp.float32),
            pltpu.VMEM((2, RG, 32, 128), jnp.float32),
            pltpu.SemaphoreType.DMA,
            pltpu.SemaphoreType.DMA,
            pltpu.SemaphoreType.DMA,
            pltpu.SemaphoreType.DMA,
        ],
    )
    def k(ind_hbm, tab_hbm, out_hbm, i0_v, i1_v, j_v, idx_v, rows_v,
          tiles_v, gsem0, gsem1, osem0, osem1):
        wid = lax.axis_index("s") * 2 + lax.axis_index("c")
        base = wid * PER_W
        lane = lax.iota(jnp.int32, 16)
        gsems = (gsem0, gsem1)
        osems = (osem0, osem1)

        pltpu.sync_copy(ind_hbm.at[pl.ds(base, PER_W)], i0_v)
        pltpu.sync_copy(ind_hbm.at[pl.ds(B + base, PER_W)], i1_v)
        for v in range(PER_W // 16):
            sl = pl.ds(v * 16, 16)
            j_v[sl] = i0_v[sl] * N2 + i1_v[sl]

        def build_and_fire(r, s):
            bh_i = r // NRG
            r1g = r % NRG
            for gi in range(RG):
                rbase = (r1g * RG + gi) * (N1 * N2)
                for v in range(GL // 16):
                    idx_v[s, gi, pl.ds(v * 16, 16)] = (
                        j_v[pl.ds(bh_i * 128 + v * 16, 16)] + rbase
                    )
            for gi in range(RG):
                pltpu.async_copy(
                    tab_hbm.at[idx_v.at[s, gi]],
                    rows_v.at[s, pl.ds(gi * GL, GL), :],
                    gsems[s],
                )

        def drain_gathers(s):
            for gi in range(RG):
                pltpu.make_async_copy(
                    tab_hbm.at[idx_v.at[s, gi]],
                    rows_v.at[s, pl.ds(gi * GL, GL), :],
                    gsems[s],
                ).wait()

        def transpose(s):
            rows2 = rows_v.at[s]
            tiles2 = tiles_v.at[s]

            @plsc.parallel_loop(0, RG * 32, unroll=8)
            def m_body(m):
                r1i = m // 32
                c0 = m % 32
                r2v = (c0 + lane) & 31
                rlv = r1i * 128 + lane
                for b8 in range(8):
                    bl0 = b8 * 16
                    val = plsc.load_gather(rows2, [rlv + bl0, r2v])
                    plsc.store_scatter(
                        tiles2.at[r1i], [r2v, lane + bl0], val)

        def out_copies(r, s):
            bh = wid * NBH + r // NRG
            r1g = r % NRG
            return [
                pltpu.make_async_copy(
                    tiles_v.at[s, r1i, pl.ds(r2h * 8, 8), :],
                    out_hbm.at[r1g * RG + r1i, r2h, bh],
                    osems[s],
                )
                for r1i in range(RG)
                for r2h in range(4)
            ]

        def fire_outs(r, s):
            for c in out_copies(r, s):
                c.start()

        def drain_outs(r, s):
            for c in out_copies(r, s):
                c.wait()

        build_and_fire(0, 0)

        def pair_body(q, carry):
            r_odd = 2 * q + 1

            @pl.when(q >= 1)
            def _():
                drain_outs(2 * q - 2, 0)

            build_and_fire(r_odd, 1)
            drain_gathers(0)
            transpose(0)
            fire_outs(2 * q, 0)

            @pl.when(q >= 1)
            def _():
                drain_outs(2 * q - 1, 1)

            @pl.when(q < NROUND // 2 - 1)
            def _():
                build_and_fire(2 * q + 2, 0)

            drain_gathers(1)
            transpose(1)
            fire_outs(2 * q + 1, 1)
            return carry

        lax.fori_loop(0, NROUND // 2, pair_body, 0)
        drain_outs(NROUND - 2, 0)
        drain_outs(NROUND - 1, 1)

    return k(ind_flat, table)


def kernel(indices, tt_core):
    ind_flat = indices.T.reshape(-1)
    table = tt_core.reshape(R1 * N1 * N2, R2)
    out6 = _sc_kernel(ind_flat, table)
    return out6.transpose(2, 4, 0, 1, 3).reshape(B, R1, R2)
